# baseline (speedup 1.0000x reference)
import functools

import jax
import jax.numpy as jnp
from jax import lax
from jax.experimental import pallas as pl
from jax.experimental.pallas import tpu as pltpu
from jax.experimental.pallas import tpu_sc as plsc

ALPHA = 1.702
LIMIT = 7.0
E = 8
H = 2048
D = 2048
T = 2048
TWO_T = 2 * T

B = 512
NB = -(-(TWO_T + E * (B - 1)) // B)
NPAD = NB * B
DJ = 512
J = D // DJ

NC = 2
NS = 16
NW = NC * NS

G_PER_W = NPAD // NW
GCH = 16
G_NCH = G_PER_W // GCH
GNB = 3

C_PER_W = T // NW
CCH = 16
C_NCH = C_PER_W // CCH

_SC_MESH = dict(core_axis_name="c", subcore_axis_name="s")


def _wid():
    return lax.axis_index("s") * NC + lax.axis_index("c")


def _gather_call(hs2d, idx3, meta):
    @functools.partial(
        pl.kernel,
        out_type=jax.ShapeDtypeStruct((NPAD, H), jnp.float32),
        mesh=plsc.VectorSubcoreMesh(**_SC_MESH),
        scratch_types=[
            pltpu.VMEM((G_NCH, GCH), jnp.int32),
            pltpu.VMEM((16,), jnp.int32),
            [pltpu.VMEM((GCH, H), jnp.float32) for _ in range(GNB)],
            [pltpu.SemaphoreType.DMA for _ in range(GNB)],
            [pltpu.SemaphoreType.DMA for _ in range(GNB)],
        ],
    )
    def gather_k(hs_hbm, idx_hbm, meta_hbm, x_hbm, idx_v, meta_v, bufs, gsems, wsems):
        base = _wid() * G_PER_W
        pltpu.sync_copy(meta_hbm, meta_v)
        n0 = meta_v[...][0]

        @pl.when(base < n0)
        def _work():
            pltpu.sync_copy(idx_hbm.at[_wid()], idx_v)
            cg = {}
            cw = {}
            for c in range(min(GNB, G_NCH)):
                cg[c] = pltpu.async_copy(
                    hs_hbm.at[idx_v.at[c]], bufs[c % GNB], gsems[c % GNB])
            for c in range(G_NCH):
                cg[c].wait()
                cw[c] = pltpu.async_copy(
                    bufs[c % GNB], x_hbm.at[pl.ds(base + c * GCH, GCH)],
                    wsems[c % GNB])
                nxt = c + GNB
                if nxt < G_NCH:
                    cw[c].wait()
                    cg[nxt] = pltpu.async_copy(
                        hs_hbm.at[idx_v.at[nxt]], bufs[c % GNB], gsems[c % GNB])
            for c in range(max(0, G_NCH - GNB), G_NCH):
                cw[c].wait()

    return gather_k(hs2d, idx3, meta)


def _combine_call(y, p0r, p1r):
    @functools.partial(
        pl.kernel,
        out_type=jax.ShapeDtypeStruct((T, H), jnp.float32),
        mesh=plsc.VectorSubcoreMesh(**_SC_MESH),
        scratch_types=[
            pltpu.VMEM((C_NCH, CCH), jnp.int32),
            pltpu.VMEM((C_NCH, CCH), jnp.int32),
            pltpu.VMEM((CCH, H), jnp.float32),
            pltpu.VMEM((CCH, H), jnp.float32),
            pltpu.SemaphoreType.DMA,
            pltpu.SemaphoreType.DMA,
        ],
    )
    def combine_k(y_hbm, p0_hbm, p1_hbm, out_hbm, i0_v, i1_v, bufa, bufb, sema, semb):
        base = _wid() * C_PER_W
        pltpu.sync_copy(p0_hbm.at[_wid()], i0_v)
        pltpu.sync_copy(p1_hbm.at[_wid()], i1_v)
        for c in range(C_NCH):
            cpa = pltpu.async_copy(y_hbm.at[i0_v.at[c]], bufa, sema)
            cpb = pltpu.async_copy(y_hbm.at[i1_v.at[c]], bufb, semb)
            cpa.wait()
            cpb.wait()
            for r in range(CCH):
                def add_row(k, _, r=r):
                    sl = pl.ds(k * 16, 16)
                    bufa[r, sl] = bufa[r, sl] + bufb[r, sl]
                    return 0
                lax.fori_loop(0, H // 16, add_row, 0, unroll=8)
            pltpu.sync_copy(bufa, out_hbm.at[pl.ds(base + c * CCH, CCH)])

    return combine_k(y, p0r, p1r)


def _ffn_body(s_ref, x_ref, wg_ref, wu_ref, bg_ref, bu_ref, w2_ref, b2_ref,
              rw_ref, y_ref, inter_ref):
    b = pl.program_id(0)
    j = pl.program_id(1)
    active = b < s_ref[NB]

    @pl.when(active & (j < J))
    def _up():
        x = x_ref[...]
        gate = lax.dot_general(x, wg_ref[0], (((1,), (1,)), ((), ())),
                               preferred_element_type=jnp.float32) + bg_ref[0]
        up = lax.dot_general(x, wu_ref[0], (((1,), (1,)), ((), ())),
                             preferred_element_type=jnp.float32) + bu_ref[0]
        gate = jnp.minimum(gate, LIMIT)
        up = jnp.clip(up, -LIMIT, LIMIT)
        glu = gate * jax.nn.sigmoid(gate * ALPHA)
        col = pl.multiple_of(j * DJ, DJ)
        inter_ref[:, pl.ds(col, DJ)] = (up + 1.0) * glu

    @pl.when(active & (j >= J))
    def _down():
        val = lax.dot_general(inter_ref[...], w2_ref[0], (((1,), (1,)), ((), ())),
                              preferred_element_type=jnp.float32)
        rw = rw_ref[0, 0, :][:, None]
        y_ref[...] = jnp.where(rw > 0.0, (val + b2_ref[0]) * rw, 0.0)

    @pl.when(jnp.logical_not(active) & (j >= J))
    def _pad():
        y_ref[...] = jnp.zeros_like(y_ref)


def _ffn_call(scalars, x, gup2, bg, bu, down_w, b2, rw3):
    jup = lambda j: jnp.minimum(j, J - 1)
    jdn = lambda j: jnp.clip(j - J, 0, J - 1)
    return pl.pallas_call(
        _ffn_body,
        grid_spec=pltpu.PrefetchScalarGridSpec(
            num_scalar_prefetch=1,
            grid=(NB, 2 * J),
            in_specs=[
                pl.BlockSpec((B, H),
                             lambda b, j, s: (jnp.minimum(b, s[NB] - 1), 0)),
                pl.BlockSpec((1, DJ, H), lambda b, j, s: (s[b], jup(j), 0)),
                pl.BlockSpec((1, DJ, H), lambda b, j, s: (s[b], jup(j), 1)),
                pl.BlockSpec((1, 1, DJ), lambda b, j, s: (s[b], 0, jup(j))),
                pl.BlockSpec((1, 1, DJ), lambda b, j, s: (s[b], 0, jup(j))),
                pl.BlockSpec((1, DJ, H), lambda b, j, s: (s[b], jdn(j), 0)),
                pl.BlockSpec((1, 1, DJ), lambda b, j, s: (s[b], 0, jdn(j))),
                pl.BlockSpec((1, 1, B), lambda b, j, s: (b, 0, 0)),
            ],
            out_specs=pl.BlockSpec((B, DJ), lambda b, j, s: (b, jdn(j))),
            scratch_shapes=[pltpu.VMEM((B, H), jnp.float32)],
        ),
        out_shape=jax.ShapeDtypeStruct((NPAD, H), jnp.float32),
    )(scalars, x, gup2, gup2, bg, bu, down_w, b2, rw3)


def kernel(hidden_states, router_indices, routing_weights, gate_up_w,
           gate_up_b, down_w, down_b):
    batch = hidden_states.shape[0]
    hs2d = hidden_states.reshape(-1, H)

    e0 = router_indices[:, 0].astype(jnp.int32)
    e1 = router_indices[:, 1].astype(jnp.int32)
    dup = e1 == e0
    ex = jnp.concatenate([e0, jnp.where(dup, E, e1)])
    tok = jnp.concatenate([jnp.arange(T, dtype=jnp.int32)] * 2)
    order = jnp.argsort(ex, stable=True)
    se = ex[order]
    st = tok[order]
    counts = jnp.bincount(se, length=E + 1)[:E].astype(jnp.int32)
    blocks_e = (counts + B - 1) // B
    block_cum = jnp.cumsum(blocks_e)
    padded_start = jnp.concatenate(
        [jnp.zeros(1, jnp.int32), block_cum[:-1]]).astype(jnp.int32) * B
    seg_start = jnp.concatenate(
        [jnp.zeros(1, jnp.int32), jnp.cumsum(counts)[:-1].astype(jnp.int32)])
    valid = se < E
    se_c = jnp.minimum(se, E - 1)
    rank = jnp.arange(TWO_T, dtype=jnp.int32) - seg_start[se_c]
    pos = jnp.where(valid, padded_start[se_c] + rank, NPAD - 1).astype(jnp.int32)

    rows_tok = jnp.zeros((NPAD,), jnp.int32).at[pos].set(st)
    w_pair = jnp.where(valid, routing_weights[st, se_c], 0.0)
    rows_w = jnp.zeros((NPAD,), jnp.float32).at[pos].set(w_pair)

    be = jnp.minimum(
        jnp.searchsorted(block_cum, jnp.arange(NB, dtype=jnp.int32),
                         side="right"), E - 1).astype(jnp.int32)
    nab = block_cum[-1].astype(jnp.int32)
    e_last = be[jnp.maximum(nab - 1, 0)]
    block_expert = jnp.where(jnp.arange(NB) < nab, be, e_last)
    scalars = jnp.concatenate([block_expert, nab[None]])

    pair_pos = jnp.zeros((TWO_T,), jnp.int32).at[order].set(pos)
    p0r = pair_pos[:T].reshape(NW, C_NCH, CCH)
    p1r = pair_pos[T:].reshape(NW, C_NCH, CCH)

    n_padded = block_cum[-1].astype(jnp.int32) * B
    meta = jnp.full((16,), n_padded, jnp.int32)
    x = _gather_call(hs2d, rows_tok.reshape(NW, G_NCH, GCH), meta)

    gup2 = gate_up_w.reshape(E, D, 2 * H)
    bg = gate_up_b[:, 0::2].reshape(E, 1, D)
    bu = gate_up_b[:, 1::2].reshape(E, 1, D)
    b2 = down_b.reshape(E, 1, H)
    rw3 = rows_w.reshape(NB, 1, B)

    y = _ffn_call(scalars, x, gup2, bg, bu, down_w, b2, rw3)

    out = _combine_call(y, p0r, p1r)
    return out.reshape(batch, T, H)


# speedup vs baseline: 1.2353x; 1.2353x over previous
import functools

import jax
import jax.numpy as jnp
from jax import lax
from jax.experimental import pallas as pl
from jax.experimental.pallas import tpu as pltpu
from jax.experimental.pallas import tpu_sc as plsc

ALPHA = 1.702
LIMIT = 7.0
E = 8
H = 2048
D = 2048
T = 2048
TWO_T = 2 * T

B = 512
NB = -(-(TWO_T + E * (B - 1)) // B)
NPAD = NB * B
DJ = 512
J = D // DJ

NC = 2
NS = 16
NW = NC * NS

G_PER_W = NPAD // NW
GCH = 16
G_NCH = G_PER_W // GCH
GNB = 3

C_PER_W = T // NW
CCH = 16
C_NCH = C_PER_W // CCH

_SC_MESH = dict(core_axis_name="c", subcore_axis_name="s")


def _wid():
    return lax.axis_index("s") * NC + lax.axis_index("c")


def _gather_call(hs2d, idx3, meta):
    @functools.partial(
        pl.kernel,
        out_type=jax.ShapeDtypeStruct((NPAD, H), jnp.float32),
        mesh=plsc.VectorSubcoreMesh(**_SC_MESH),
        scratch_types=[
            pltpu.VMEM((G_NCH, GCH), jnp.int32),
            pltpu.VMEM((16,), jnp.int32),
            [pltpu.VMEM((GCH, H), jnp.float32) for _ in range(GNB)],
            [pltpu.SemaphoreType.DMA for _ in range(GNB)],
            [pltpu.SemaphoreType.DMA for _ in range(GNB)],
        ],
    )
    def gather_k(hs_hbm, idx_hbm, meta_hbm, x_hbm, idx_v, meta_v, bufs, gsems, wsems):
        base = _wid() * G_PER_W
        pltpu.sync_copy(meta_hbm, meta_v)
        n0 = meta_v[...][0]

        @pl.when(base < n0)
        def _work():
            pltpu.sync_copy(idx_hbm.at[_wid()], idx_v)
            cg = {}
            cw = {}
            for c in range(min(GNB, G_NCH)):
                cg[c] = pltpu.async_copy(
                    hs_hbm.at[idx_v.at[c]], bufs[c % GNB], gsems[c % GNB])
            for c in range(G_NCH):
                cg[c].wait()
                cw[c] = pltpu.async_copy(
                    bufs[c % GNB], x_hbm.at[pl.ds(base + c * GCH, GCH)],
                    wsems[c % GNB])
                nxt = c + GNB
                if nxt < G_NCH:
                    cw[c].wait()
                    cg[nxt] = pltpu.async_copy(
                        hs_hbm.at[idx_v.at[nxt]], bufs[c % GNB], gsems[c % GNB])
            for c in range(max(0, G_NCH - GNB), G_NCH):
                cw[c].wait()

    return gather_k(hs2d, idx3, meta)


def _combine_call(y, p0r, p1r):
    @functools.partial(
        pl.kernel,
        out_type=jax.ShapeDtypeStruct((T, H), jnp.float32),
        mesh=plsc.VectorSubcoreMesh(**_SC_MESH),
        scratch_types=[
            pltpu.VMEM((C_NCH, CCH), jnp.int32),
            pltpu.VMEM((C_NCH, CCH), jnp.int32),
            pltpu.VMEM((CCH, H), jnp.float32),
            pltpu.VMEM((CCH, H), jnp.float32),
            pltpu.SemaphoreType.DMA,
            pltpu.SemaphoreType.DMA,
        ],
    )
    def combine_k(y_hbm, p0_hbm, p1_hbm, out_hbm, i0_v, i1_v, bufa, bufb, sema, semb):
        base = _wid() * C_PER_W
        pltpu.sync_copy(p0_hbm.at[_wid()], i0_v)
        pltpu.sync_copy(p1_hbm.at[_wid()], i1_v)
        for c in range(C_NCH):
            cpa = pltpu.async_copy(y_hbm.at[i0_v.at[c]], bufa, sema)
            cpb = pltpu.async_copy(y_hbm.at[i1_v.at[c]], bufb, semb)
            cpa.wait()
            cpb.wait()
            for r in range(CCH):
                def add_row(k, _, r=r):
                    sl = pl.ds(k * 16, 16)
                    bufa[r, sl] = bufa[r, sl] + bufb[r, sl]
                    return 0
                lax.fori_loop(0, H // 16, add_row, 0, unroll=8)
            pltpu.sync_copy(bufa, out_hbm.at[pl.ds(base + c * CCH, CCH)])

    return combine_k(y, p0r, p1r)


def _ffn_body(s_ref, x_ref, wgu_ref, bgu_ref, w2_ref, b2_ref,
              rw_ref, y_ref, acc_ref):
    b = pl.program_id(0)
    j = pl.program_id(1)
    active = b < s_ref[NB]

    @pl.when(active & (j == 0))
    def _zero():
        acc_ref[...] = jnp.zeros_like(acc_ref)

    @pl.when(active)
    def _compute():
        x = x_ref[...]
        wgu = wgu_ref[0]
        gate = lax.dot_general(x, wgu[:, :H], (((1,), (1,)), ((), ())),
                               preferred_element_type=jnp.float32) + bgu_ref[0, 0]
        up = lax.dot_general(x, wgu[:, H:], (((1,), (1,)), ((), ())),
                             preferred_element_type=jnp.float32) + bgu_ref[0, 1]
        gate = jnp.minimum(gate, LIMIT)
        up = jnp.clip(up, -LIMIT, LIMIT)
        glu = gate * jax.nn.sigmoid(gate * ALPHA)
        inter = (up + 1.0) * glu
        acc_ref[...] += lax.dot_general(inter, w2_ref[0], (((1,), (1,)), ((), ())),
                                        preferred_element_type=jnp.float32)

    @pl.when(j == J - 1)
    def _emit():
        rw = rw_ref[0, 0, :][:, None]
        y_ref[...] = jnp.where(rw > 0.0, (acc_ref[...] + b2_ref[0]) * rw, 0.0)


def _ffn_call(scalars, x, gup2, bgu, down_w, b2, rw3):
    return pl.pallas_call(
        _ffn_body,
        grid_spec=pltpu.PrefetchScalarGridSpec(
            num_scalar_prefetch=1,
            grid=(NB, J),
            in_specs=[
                pl.BlockSpec((B, H),
                             lambda b, j, s: (jnp.minimum(b, s[NB] - 1), 0)),
                pl.BlockSpec((1, DJ, 2 * H),
                             lambda b, j, s: (s[b], jnp.where(b < s[NB], j, 0), 0)),
                pl.BlockSpec((1, 2, DJ),
                             lambda b, j, s: (s[b], 0, jnp.where(b < s[NB], j, 0))),
                pl.BlockSpec((1, H, DJ),
                             lambda b, j, s: (s[b], 0, jnp.where(b < s[NB], j, 0))),
                pl.BlockSpec((1, 1, H), lambda b, j, s: (s[b], 0, 0)),
                pl.BlockSpec((1, 1, B), lambda b, j, s: (b, 0, 0)),
            ],
            out_specs=pl.BlockSpec((B, H), lambda b, j, s: (b, 0)),
            scratch_shapes=[pltpu.VMEM((B, H), jnp.float32)],
        ),
        out_shape=jax.ShapeDtypeStruct((NPAD, H), jnp.float32),
    )(scalars, x, gup2, bgu, down_w, b2, rw3)


def kernel(hidden_states, router_indices, routing_weights, gate_up_w,
           gate_up_b, down_w, down_b):
    batch = hidden_states.shape[0]
    hs2d = hidden_states.reshape(-1, H)

    e0 = router_indices[:, 0].astype(jnp.int32)
    e1 = router_indices[:, 1].astype(jnp.int32)
    dup = e1 == e0
    ex = jnp.concatenate([e0, jnp.where(dup, E, e1)])
    tok = jnp.concatenate([jnp.arange(T, dtype=jnp.int32)] * 2)
    order = jnp.argsort(ex, stable=True)
    se = ex[order]
    st = tok[order]
    counts = jnp.bincount(se, length=E + 1)[:E].astype(jnp.int32)
    blocks_e = (counts + B - 1) // B
    block_cum = jnp.cumsum(blocks_e)
    padded_start = jnp.concatenate(
        [jnp.zeros(1, jnp.int32), block_cum[:-1]]).astype(jnp.int32) * B
    seg_start = jnp.concatenate(
        [jnp.zeros(1, jnp.int32), jnp.cumsum(counts)[:-1].astype(jnp.int32)])
    valid = se < E
    se_c = jnp.minimum(se, E - 1)
    rank = jnp.arange(TWO_T, dtype=jnp.int32) - seg_start[se_c]
    pos = jnp.where(valid, padded_start[se_c] + rank, NPAD - 1).astype(jnp.int32)

    rows_tok = jnp.zeros((NPAD,), jnp.int32).at[pos].set(st)
    w_pair = jnp.where(valid, routing_weights[st, se_c], 0.0)
    rows_w = jnp.zeros((NPAD,), jnp.float32).at[pos].set(w_pair)

    be = jnp.minimum(
        jnp.searchsorted(block_cum, jnp.arange(NB, dtype=jnp.int32),
                         side="right"), E - 1).astype(jnp.int32)
    nab = block_cum[-1].astype(jnp.int32)
    e_last = be[jnp.maximum(nab - 1, 0)]
    block_expert = jnp.where(jnp.arange(NB) < nab, be, e_last)
    scalars = jnp.concatenate([block_expert, nab[None]])

    pair_pos = jnp.zeros((TWO_T,), jnp.int32).at[order].set(pos)
    p0r = pair_pos[:T].reshape(NW, C_NCH, CCH)
    p1r = pair_pos[T:].reshape(NW, C_NCH, CCH)

    n_padded = block_cum[-1].astype(jnp.int32) * B
    meta = jnp.full((16,), n_padded, jnp.int32)
    x = _gather_call(hs2d, rows_tok.reshape(NW, G_NCH, GCH), meta)

    gup2 = gate_up_w.reshape(E, D, 2 * H)
    bgu = jnp.stack([gate_up_b[:, 0::2], gate_up_b[:, 1::2]], axis=1)
    b2 = down_b.reshape(E, 1, H)
    rw3 = rows_w.reshape(NB, 1, B)

    y = _ffn_call(scalars, x, gup2, bgu, down_w, b2, rw3)

    out = _combine_call(y, p0r, p1r)
    return out.reshape(batch, T, H)
